# Initial kernel scaffold; baseline (speedup 1.0000x reference)
import jax
import jax.numpy as jnp
from jax.experimental import pallas as pl


def kernel(u_pos, v_pos, v_neg, batch_size, U_emb, V_emb):
    raise NotImplementedError("write your pallas kernel here")


# speedup vs baseline: 5.2257x; 5.2257x over previous
import functools

import jax
import jax.numpy as jnp
from jax import lax
from jax.experimental import pallas as pl
from jax.experimental.pallas import tpu as pltpu
from jax.experimental.pallas import tpu_sc as plsc

NC = 2
NS = 16
LANES = 16
NW = NC * NS

BATCH = 16384
DIM = 64
NNEG = 20
KD = DIM // LANES

BPW = BATCH // NW
CHUNK = 64
NCHUNK = BPW // CHUNK


def _sc_scores_body(Uemb, Vemb, upos, vpos, vneg3,
                    spos_out, sneg_out,
                    uidx, vidx, negidx, urows, vrows, negrows,
                    spos_acc, sneg_acc, sem):
    cid = lax.axis_index("c")
    sid = lax.axis_index("s")
    wid = cid * NS + sid
    wbase = wid * BPW

    def chunk_body(ch, _):
        base = wbase + ch * CHUNK
        gchunk = wid * NCHUNK + ch
        pltpu.sync_copy(upos.at[pl.ds(base, CHUNK)], uidx)
        pltpu.sync_copy(vpos.at[pl.ds(base, CHUNK)], vidx)
        pltpu.sync_copy(vneg3.at[gchunk], negidx)
        cps = [pltpu.async_copy(Uemb.at[uidx], urows, sem),
               pltpu.async_copy(Vemb.at[vidx], vrows, sem)]
        for j in range(NNEG):
            cps.append(pltpu.async_copy(Vemb.at[negidx.at[j]],
                                        negrows.at[pl.ds(j * CHUNK, CHUNK)],
                                        sem))
        for cp in cps:
            cp.wait()

        def item_body(b, _):
            u = [urows[b, pl.ds(k * LANES, LANES)] for k in range(KD)]
            v = [vrows[b, pl.ds(k * LANES, LANES)] for k in range(KD)]
            p = u[0] * v[0]
            for k in range(1, KD):
                p = p + u[k] * v[k]
            acc = [negrows[b, pl.ds(k * LANES, LANES)] for k in range(KD)]
            for j in range(1, NNEG):
                r = j * CHUNK + b
                for k in range(KD):
                    acc[k] = acc[k] + negrows[r, pl.ds(k * LANES, LANES)]
            q = acc[0] * u[0]
            for k in range(1, KD):
                q = q + acc[k] * u[k]
            row = ch * CHUNK + b
            spos_acc[row, :] = p
            sneg_acc[row, :] = q
            return 0

        lax.fori_loop(0, CHUNK, item_body, 0)
        return 0

    lax.fori_loop(0, NCHUNK, chunk_body, 0)
    pltpu.sync_copy(spos_acc, spos_out.at[pl.ds(wbase, BPW), :])
    pltpu.sync_copy(sneg_acc, sneg_out.at[pl.ds(wbase, BPW), :])


@functools.cache
def _sc_scores():
  return pl.kernel(
    _sc_scores_body,
    out_type=(jax.ShapeDtypeStruct((BATCH, LANES), jnp.float32),
              jax.ShapeDtypeStruct((BATCH, LANES), jnp.float32)),
    mesh=plsc.VectorSubcoreMesh(core_axis_name="c", subcore_axis_name="s",
                                num_cores=NC, num_subcores=NS),
    scratch_types=(
        pltpu.VMEM((CHUNK,), jnp.int32),
        pltpu.VMEM((CHUNK,), jnp.int32),
        pltpu.VMEM((NNEG, CHUNK), jnp.int32),
        pltpu.VMEM((CHUNK, DIM), jnp.float32),
        pltpu.VMEM((CHUNK, DIM), jnp.float32),
        pltpu.VMEM((NNEG * CHUNK, DIM), jnp.float32),
        pltpu.VMEM((BPW, LANES), jnp.float32),
        pltpu.VMEM((BPW, LANES), jnp.float32),
        pltpu.SemaphoreType.DMA,
    ),
    compiler_params=pltpu.CompilerParams(use_tc_tiling_on_sc=False),
  )


def _finish_body(spos_ref, sneg_ref, out_ref):
    sp = jnp.sum(spos_ref[...], axis=1)
    sn = -jnp.sum(sneg_ref[...], axis=1)

    def logsig(x):
        return jnp.minimum(x, 0.0) - jnp.log1p(jnp.exp(-jnp.abs(x)))

    loss = logsig(sp) + logsig(sn)
    out_ref[0, 0] = -jnp.sum(loss) / BATCH


_finish = pl.pallas_call(
    _finish_body,
    out_shape=jax.ShapeDtypeStruct((1, 1), jnp.float32),
    out_specs=pl.BlockSpec(memory_space=pltpu.SMEM),
)


@jax.jit
def kernel(u_pos, v_pos, v_neg, batch_size, U_emb, V_emb):
    del batch_size
    upos = u_pos.reshape(BATCH)
    vpos = v_pos.reshape(BATCH)
    vneg3 = jnp.transpose(v_neg.reshape(NW * NCHUNK, CHUNK, NNEG), (0, 2, 1))
    spos, sneg = _sc_scores()(U_emb, V_emb, upos, vpos, vneg3)
    out = _finish(spos, sneg)
    return out[0, 0]


# speedup vs baseline: 5.3281x; 1.0196x over previous
import functools

import jax
import jax.numpy as jnp
from jax import lax
from jax.experimental import pallas as pl
from jax.experimental.pallas import tpu as pltpu
from jax.experimental.pallas import tpu_sc as plsc

NC = 2
NS = 16
LANES = 16
NW = NC * NS

BATCH = 16384
DIM = 64
NNEG = 20
KD = DIM // LANES

BPW = BATCH // NW
CHUNK = 64
NCHUNK = BPW // CHUNK
SLEN = 128
NSTREAM = CHUNK * NNEG // SLEN


def _sc_scores_body(Uemb, Vemb, upos, vpos, vneg3,
                    spos_out, sneg_out,
                    uidx, vidx, negidx, urows, vrows, negrows,
                    spos_acc, sneg_acc, sem):
    cid = lax.axis_index("c")
    sid = lax.axis_index("s")
    wid = cid * NS + sid
    wbase = wid * BPW

    def chunk_body(ch, _):
        base = wbase + ch * CHUNK
        gchunk = wid * NCHUNK + ch
        pltpu.sync_copy(upos.at[pl.ds(base, CHUNK)], uidx)
        pltpu.sync_copy(vpos.at[pl.ds(base, CHUNK)], vidx)
        pltpu.sync_copy(vneg3.at[gchunk], negidx)
        cps = [pltpu.async_copy(Uemb.at[uidx], urows, sem),
               pltpu.async_copy(Vemb.at[vidx], vrows, sem)]
        for s in range(NSTREAM):
            cps.append(pltpu.async_copy(Vemb.at[negidx.at[s]],
                                        negrows.at[pl.ds(s * SLEN, SLEN)],
                                        sem))
        for cp in cps:
            cp.wait()

        def item_body(b, _):
            u = [urows[b, pl.ds(k * LANES, LANES)] for k in range(KD)]
            v = [vrows[b, pl.ds(k * LANES, LANES)] for k in range(KD)]
            p = u[0] * v[0]
            for k in range(1, KD):
                p = p + u[k] * v[k]
            r0 = b * NNEG
            acc = [negrows[r0, pl.ds(k * LANES, LANES)] for k in range(KD)]
            for j in range(1, NNEG):
                for k in range(KD):
                    acc[k] = acc[k] + negrows[r0 + j, pl.ds(k * LANES, LANES)]
            q = acc[0] * u[0]
            for k in range(1, KD):
                q = q + acc[k] * u[k]
            off = (ch * CHUNK + b) * LANES
            spos_acc[pl.ds(off, LANES)] = p
            sneg_acc[pl.ds(off, LANES)] = q
            return 0

        lax.fori_loop(0, CHUNK, item_body, 0)
        return 0

    lax.fori_loop(0, NCHUNK, chunk_body, 0)
    pltpu.sync_copy(spos_acc, spos_out.at[pl.ds(wbase * LANES, BPW * LANES)])
    pltpu.sync_copy(sneg_acc, sneg_out.at[pl.ds(wbase * LANES, BPW * LANES)])


@functools.cache
def _sc_scores():
  return pl.kernel(
    _sc_scores_body,
    out_type=(jax.ShapeDtypeStruct((BATCH * LANES,), jnp.float32),
              jax.ShapeDtypeStruct((BATCH * LANES,), jnp.float32)),
    mesh=plsc.VectorSubcoreMesh(core_axis_name="c", subcore_axis_name="s",
                                num_cores=NC, num_subcores=NS),
    scratch_types=(
        pltpu.VMEM((CHUNK,), jnp.int32),
        pltpu.VMEM((CHUNK,), jnp.int32),
        pltpu.VMEM((NSTREAM, SLEN), jnp.int32),
        pltpu.VMEM((CHUNK, DIM), jnp.float32),
        pltpu.VMEM((CHUNK, DIM), jnp.float32),
        pltpu.VMEM((NNEG * CHUNK, DIM), jnp.float32),
        pltpu.VMEM((BPW * LANES,), jnp.float32),
        pltpu.VMEM((BPW * LANES,), jnp.float32),
        pltpu.SemaphoreType.DMA,
    ),
    compiler_params=pltpu.CompilerParams(use_tc_tiling_on_sc=False),
  )


def _finish_body(spos_ref, sneg_ref, out_ref):
    il = lax.broadcasted_iota(jnp.int32, (128, 8), 0)
    ig = lax.broadcasted_iota(jnp.int32, (128, 8), 1)
    mask = (il // LANES == ig).astype(jnp.float32)
    sp = jnp.dot(spos_ref[...], mask, preferred_element_type=jnp.float32)
    sn = -jnp.dot(sneg_ref[...], mask, preferred_element_type=jnp.float32)

    def logsig(x):
        return jnp.minimum(x, 0.0) - jnp.log1p(jnp.exp(-jnp.abs(x)))

    loss = logsig(sp) + logsig(sn)
    out_ref[0, 0] = -jnp.sum(loss) / BATCH


_finish = pl.pallas_call(
    _finish_body,
    out_shape=jax.ShapeDtypeStruct((1, 1), jnp.float32),
    out_specs=pl.BlockSpec(memory_space=pltpu.SMEM),
)


@jax.jit
def kernel(u_pos, v_pos, v_neg, batch_size, U_emb, V_emb):
    del batch_size
    upos = u_pos.reshape(BATCH)
    vpos = v_pos.reshape(BATCH)
    vneg3 = v_neg.reshape(NW * NCHUNK, NSTREAM, SLEN)
    spos, sneg = _sc_scores()(U_emb, V_emb, upos, vpos, vneg3)
    out = _finish(spos.reshape(BATCH * LANES // 128, 128),
                  sneg.reshape(BATCH * LANES // 128, 128))
    return out[0, 0]
